# baseline (speedup 1.0000x reference)
import functools

import jax
import jax.numpy as jnp
from jax import lax
from jax.experimental import pallas as pl
from jax.experimental.pallas import tpu as pltpu
from jax.experimental.pallas import tpu_sc as plsc

NC, NS = 2, 16
NW = NC * NS
CHUNK = 128
NBUF = 4
PADW = 128
DIM = 64
L = 16


@functools.partial(jax.jit, static_argnames=("nchunks",))
def _sc_lookup(ids, table_p, *, nchunks):
    total = NW * nchunks * CHUNK
    ngroups = nchunks // NBUF
    mesh = plsc.VectorSubcoreMesh(core_axis_name="c", subcore_axis_name="s")

    @functools.partial(
        pl.kernel,
        out_type=jax.ShapeDtypeStruct((total // 2, PADW), jnp.float32),
        mesh=mesh,
        scratch_types=[
            pltpu.VMEM((nchunks, CHUNK), jnp.int32),
            pltpu.VMEM((NBUF, CHUNK, PADW), jnp.float32),
            pltpu.VMEM((NBUF, CHUNK // 2, PADW), jnp.float32),
            [pltpu.SemaphoreType.DMA] * NBUF,
            [pltpu.SemaphoreType.DMA] * NBUF,
        ],
        compiler_params=pltpu.CompilerParams(use_tc_tiling_on_sc=True),
    )
    def body(ids_hbm, table_hbm, out_hbm, idx_v, rows_v, pack_v, gsems, ssems):
        wid = lax.axis_index("s") * NC + lax.axis_index("c")
        pltpu.sync_copy(ids_hbm.at[wid], idx_v)
        rowbase = wid * (nchunks * CHUNK // 2)

        def gather_start(c, b):
            pltpu.async_copy(table_hbm.at[idx_v.at[c]], rows_v.at[b], gsems[b])

        def gather_wait(b):
            pltpu.make_async_copy(
                table_hbm.at[idx_v.at[0]], rows_v.at[b], gsems[b]
            ).wait()

        def pack(b):
            for j in range(CHUNK // 2):
                for k in range(DIM // L):
                    pack_v[b, j, pl.ds(k * L, L)] = (
                        rows_v[b, 2 * j, pl.ds(k * L, L)]
                    )
                    pack_v[b, j, pl.ds(DIM + k * L, L)] = (
                        rows_v[b, 2 * j + 1, pl.ds(k * L, L)]
                    )

        def store_start(c, b):
            pltpu.async_copy(
                pack_v.at[b],
                out_hbm.at[pl.ds(rowbase + c * (CHUNK // 2), CHUNK // 2)],
                ssems[b],
            )

        def store_wait(b):
            pltpu.make_async_copy(
                pack_v.at[b], out_hbm.at[pl.ds(rowbase, CHUNK // 2)], ssems[b]
            ).wait()

        for b in range(NBUF):
            gather_start(b, b)

        def outer(g, carry):
            for b in range(NBUF):
                gather_wait(b)
                pack(b)
                store_start(g * NBUF + b, b)
            for b in range(NBUF):
                store_wait(b)
                gather_start((g + 1) * NBUF + b, b)
            return carry

        lax.fori_loop(0, ngroups - 1, outer, 0, unroll=False)

        for b in range(NBUF):
            gather_wait(b)
            pack(b)
            store_start((ngroups - 1) * NBUF + b, b)
        for b in range(NBUF):
            store_wait(b)

    return body(ids, table_p)


def kernel(input_ids, table):
    n_rows, n_cols = input_ids.shape
    total = n_rows * n_cols
    dim = table.shape[1]
    nchunks = total // (NW * CHUNK)
    ids = input_ids.reshape(NW, nchunks, CHUNK).astype(jnp.int32)
    table_p = jnp.pad(table, ((0, 0), (0, PADW - dim)))
    out = _sc_lookup(ids, table_p, nchunks=nchunks)
    return out.reshape(n_rows, n_cols, dim)


# speedup vs baseline: 1.3298x; 1.3298x over previous
import functools

import jax
import jax.numpy as jnp
from jax import lax
from jax.experimental import pallas as pl
from jax.experimental.pallas import tpu as pltpu
from jax.experimental.pallas import tpu_sc as plsc

NC, NS = 2, 16
NW = NC * NS
CHUNK = 128
NBUF = 4
PADW = 128


@functools.partial(jax.jit, static_argnames=("nchunks",))
def _sc_lookup(ids, table_p, *, nchunks):
    total = NW * nchunks * CHUNK
    ngroups = nchunks // NBUF
    mesh = plsc.VectorSubcoreMesh(core_axis_name="c", subcore_axis_name="s")

    @functools.partial(
        pl.kernel,
        out_type=jax.ShapeDtypeStruct((total, PADW), jnp.float32),
        mesh=mesh,
        scratch_types=[
            pltpu.VMEM((nchunks, CHUNK), jnp.int32),
            pltpu.VMEM((NBUF, CHUNK, PADW), jnp.float32),
            [pltpu.SemaphoreType.DMA] * NBUF,
            [pltpu.SemaphoreType.DMA] * NBUF,
        ],
        compiler_params=pltpu.CompilerParams(use_tc_tiling_on_sc=True),
    )
    def body(ids_hbm, table_hbm, out_hbm, idx_v, rows_v, gsems, ssems):
        wid = lax.axis_index("s") * NC + lax.axis_index("c")
        pltpu.sync_copy(ids_hbm.at[wid], idx_v)
        rowbase = wid * (nchunks * CHUNK)

        def gather_start(c, b):
            pltpu.async_copy(table_hbm.at[idx_v.at[c]], rows_v.at[b], gsems[b])

        def gather_wait(b):
            pltpu.make_async_copy(
                table_hbm.at[idx_v.at[0]], rows_v.at[b], gsems[b]
            ).wait()

        def store_start(c, b):
            pltpu.async_copy(
                rows_v.at[b],
                out_hbm.at[pl.ds(rowbase + c * CHUNK, CHUNK)],
                ssems[b],
            )

        def store_wait(b):
            pltpu.make_async_copy(
                rows_v.at[b], out_hbm.at[pl.ds(rowbase, CHUNK)], ssems[b]
            ).wait()

        for b in range(NBUF):
            gather_start(b, b)

        def outer(g, carry):
            for b in range(NBUF):
                gather_wait(b)
                store_start(g * NBUF + b, b)
            for b in range(NBUF):
                store_wait(b)
                gather_start((g + 1) * NBUF + b, b)
            return carry

        lax.fori_loop(0, ngroups - 1, outer, 0, unroll=False)

        for b in range(NBUF):
            gather_wait(b)
            store_start((ngroups - 1) * NBUF + b, b)
        for b in range(NBUF):
            store_wait(b)

    return body(ids, table_p)


def kernel(input_ids, table):
    n_rows, n_cols = input_ids.shape
    total = n_rows * n_cols
    dim = table.shape[1]
    nchunks = total // (NW * CHUNK)
    ids = input_ids.reshape(NW, nchunks, CHUNK).astype(jnp.int32)
    table_p = jnp.pad(table, ((0, 0), (0, PADW - dim)))
    out = _sc_lookup(ids, table_p, nchunks=nchunks)
    return out[:, :dim].reshape(n_rows, n_cols, dim)
